# Initial kernel scaffold; baseline (speedup 1.0000x reference)
import jax
import jax.numpy as jnp
from jax.experimental import pallas as pl


def kernel(gt_boxes, gt_labels, priors_center):
    raise NotImplementedError("write your pallas kernel here")


# speedup vs baseline: 53.8451x; 53.8451x over previous
import jax
import jax.numpy as jnp
from jax import lax
from jax.experimental import pallas as pl
from jax.experimental.pallas import tpu as pltpu

CENTER_VARIANCE = 0.1
SIZE_VARIANCE = 0.2
IOU_THRESHOLD = 0.5
EPS = 1e-5
P_REAL = 8732
P_PAD = 8960
G = 64


def _ssd_kernel(gt_ref, vals_ref, pr_ref, loc_ref, lab_ref):
    gt = gt_ref[0]
    vals = vals_ref[0]
    pcx = pr_ref[0:1, :]
    pcy = pr_ref[1:2, :]
    pw = pr_ref[2:3, :]
    ph = pr_ref[3:4, :]
    px1 = pcx - pw * 0.5
    py1 = pcy - ph * 0.5
    px2 = pcx + pw * 0.5
    py2 = pcy + ph * 0.5
    gx1 = gt[:, 0:1]
    gy1 = gt[:, 1:2]
    gx2 = gt[:, 2:3]
    gy2 = gt[:, 3:4]
    ow = jnp.maximum(jnp.minimum(gx2, px2) - jnp.maximum(gx1, px1), 0.0)
    oh = jnp.maximum(jnp.minimum(gy2, py2) - jnp.maximum(gy1, py1), 0.0)
    overlap = ow * oh
    ga = jnp.maximum(gx2 - gx1, 0.0) * jnp.maximum(gy2 - gy1, 0.0)
    pa = jnp.maximum(px2 - px1, 0.0) * jnp.maximum(py2 - py1, 0.0)
    iou = overlap / (ga + pa - overlap + EPS)
    p_iota = lax.broadcasted_iota(jnp.int32, (G, P_PAD), 1)
    g_iota = lax.broadcasted_iota(jnp.int32, (G, P_PAD), 0)
    iou = jnp.where(p_iota < P_REAL, iou, -1.0)
    best = jnp.max(iou, axis=0, keepdims=True)
    bt_idx = jnp.min(jnp.where(iou == best, g_iota, G), axis=0, keepdims=True)
    bp_val = jnp.max(iou, axis=1, keepdims=True)
    bppt = jnp.min(jnp.where(iou == bp_val, p_iota, P_PAD), axis=1, keepdims=True)
    forced = jnp.max(jnp.where(bppt == p_iota, g_iota, -1), axis=0, keepdims=True)
    new_idx = jnp.where(forced >= 0, forced, bt_idx)
    best = jnp.where(forced >= 0, 2.0, best)
    onehot = (g_iota == new_idx).astype(jnp.float32)
    out8 = lax.dot_general(vals, onehot, (((1,), (0,)), ((), ())),
                           preferred_element_type=jnp.float32)
    x1, y1, x2, y2 = out8[0:1], out8[1:2], out8[2:3], out8[3:4]
    labf = out8[4:5]
    cx = (x1 + x2) / 2
    cy = (y1 + y2) / 2
    w = x2 - x1
    h = y2 - y1
    loc_x = (cx - pcx) / (pw * CENTER_VARIANCE)
    loc_y = (cy - pcy) / (ph * CENTER_VARIANCE)
    loc_w = jnp.log(w / pw) / SIZE_VARIANCE
    loc_h = jnp.log(h / ph) / SIZE_VARIANCE
    loc_ref[0] = jnp.concatenate([loc_x, loc_y, loc_w, loc_h], axis=0)
    lab_ref[0] = jnp.where(best < IOU_THRESHOLD, 0, labf.astype(jnp.int32))


def kernel(gt_boxes, gt_labels, priors_center):
    B = gt_boxes.shape[0]
    gtT = jnp.transpose(gt_boxes, (0, 2, 1)).astype(jnp.float32)
    labf = gt_labels.astype(jnp.float32)[:, None, :]
    zeros = jnp.zeros((B, 3, G), jnp.float32)
    vals = jnp.concatenate([gtT, labf, zeros], axis=1)
    prT = jnp.transpose(priors_center)
    prT = jnp.pad(prT, ((0, 0), (0, P_PAD - P_REAL)), constant_values=1.0)
    loc, lab = pl.pallas_call(
        _ssd_kernel,
        grid=(B,),
        in_specs=[
            pl.BlockSpec((1, G, 4), lambda b: (b, 0, 0)),
            pl.BlockSpec((1, 8, G), lambda b: (b, 0, 0)),
            pl.BlockSpec((4, P_PAD), lambda b: (0, 0)),
        ],
        out_specs=[
            pl.BlockSpec((1, 4, P_PAD), lambda b: (b, 0, 0)),
            pl.BlockSpec((1, 1, P_PAD), lambda b: (b, 0, 0)),
        ],
        out_shape=[
            jax.ShapeDtypeStruct((B, 4, P_PAD), jnp.float32),
            jax.ShapeDtypeStruct((B, 1, P_PAD), jnp.int32),
        ],
        compiler_params=pltpu.CompilerParams(
            dimension_semantics=("parallel",),
            vmem_limit_bytes=64 * 1024 * 1024,
        ),
        name="ssd_target_transform",
    )(gt_boxes, vals, prT)
    locations = jnp.transpose(loc, (0, 2, 1))[:, :P_REAL, :]
    labels = lab[:, 0, :P_REAL]
    return locations, labels


# speedup vs baseline: 56.5358x; 1.0500x over previous
import jax
import jax.numpy as jnp
from jax import lax
from jax.experimental import pallas as pl
from jax.experimental.pallas import tpu as pltpu

CENTER_VARIANCE = 0.1
SIZE_VARIANCE = 0.2
IOU_THRESHOLD = 0.5
EPS = 1e-5
P_REAL = 8732
P_PAD = 8960
G = 64


def _ssd_kernel(gt_ref, vals_ref, tb_ref, loc_ref, lab_ref):
    gt = gt_ref[0]
    vals = vals_ref[0]
    pcx = tb_ref[0:1, :]
    pcy = tb_ref[1:2, :]
    pw = tb_ref[2:3, :]
    ph = tb_ref[3:4, :]
    px1 = tb_ref[4:5, :]
    py1 = tb_ref[5:6, :]
    px2 = tb_ref[6:7, :]
    py2 = tb_ref[7:8, :]
    pa = tb_ref[8:9, :]
    pwv = tb_ref[9:10, :]
    phv = tb_ref[10:11, :]
    gx1 = gt[:, 0:1]
    gy1 = gt[:, 1:2]
    gx2 = gt[:, 2:3]
    gy2 = gt[:, 3:4]
    ow = jnp.maximum(jnp.minimum(gx2, px2) - jnp.maximum(gx1, px1), 0.0)
    oh = jnp.maximum(jnp.minimum(gy2, py2) - jnp.maximum(gy1, py1), 0.0)
    overlap = ow * oh
    ga = jnp.maximum(gx2 - gx1, 0.0) * jnp.maximum(gy2 - gy1, 0.0)
    iou = overlap / (ga + pa - overlap + EPS)
    p_iota = tb_ref[11:12, :]
    g_iota = lax.broadcasted_iota(jnp.int32, (G, 1), 0).astype(jnp.float32)
    best = jnp.max(iou, axis=0, keepdims=True)
    bt_idx = jnp.min(jnp.where(iou == best, g_iota, float(G)),
                     axis=0, keepdims=True)
    bp_val = jnp.max(iou, axis=1, keepdims=True)
    bppt = jnp.min(jnp.where(iou == bp_val, p_iota, float(P_PAD)),
                   axis=1, keepdims=True)
    forced = jnp.max(jnp.where(bppt == p_iota, g_iota, -1.0),
                     axis=0, keepdims=True)
    new_idx = jnp.where(forced >= 0.0, forced, bt_idx)
    best = jnp.where(forced >= 0.0, 2.0, best)
    onehot = (g_iota == new_idx).astype(jnp.float32)
    out8 = lax.dot_general(vals, onehot, (((1,), (0,)), ((), ())),
                           preferred_element_type=jnp.float32)
    x1, y1, x2, y2 = out8[0:1], out8[1:2], out8[2:3], out8[3:4]
    labf = out8[4:5]
    cx = (x1 + x2) / 2
    cy = (y1 + y2) / 2
    w = x2 - x1
    h = y2 - y1
    loc_x = (cx - pcx) / pwv
    loc_y = (cy - pcy) / phv
    loc_w = jnp.log(w / pw) / SIZE_VARIANCE
    loc_h = jnp.log(h / ph) / SIZE_VARIANCE
    loc_ref[0] = jnp.concatenate([loc_x, loc_y, loc_w, loc_h], axis=0)
    lab_ref[0] = jnp.where(best < IOU_THRESHOLD, 0, labf.astype(jnp.int32))


def kernel(gt_boxes, gt_labels, priors_center):
    B = gt_boxes.shape[0]
    gtT = jnp.transpose(gt_boxes, (0, 2, 1)).astype(jnp.float32)
    labf = gt_labels.astype(jnp.float32)[:, None, :]
    zeros = jnp.zeros((B, 3, G), jnp.float32)
    vals = jnp.concatenate([gtT, labf, zeros], axis=1)
    pad = jnp.tile(jnp.array([[-10.0, -10.0, 1.0, 1.0]], jnp.float32),
                   (P_PAD - P_REAL, 1))
    pc = jnp.concatenate([priors_center.astype(jnp.float32), pad], axis=0)
    pcx, pcy, pw, ph = pc[:, 0], pc[:, 1], pc[:, 2], pc[:, 3]
    px1 = pcx - pw / 2
    py1 = pcy - ph / 2
    px2 = pcx + pw / 2
    py2 = pcy + ph / 2
    pa = jnp.maximum(px2 - px1, 0.0) * jnp.maximum(py2 - py1, 0.0)
    table = jnp.stack([pcx, pcy, pw, ph, px1, py1, px2, py2, pa,
                       pw * CENTER_VARIANCE, ph * CENTER_VARIANCE,
                       jnp.arange(P_PAD, dtype=jnp.float32)], axis=0)
    loc, lab = pl.pallas_call(
        _ssd_kernel,
        grid=(B,),
        in_specs=[
            pl.BlockSpec((1, G, 4), lambda b: (b, 0, 0)),
            pl.BlockSpec((1, 8, G), lambda b: (b, 0, 0)),
            pl.BlockSpec((12, P_PAD), lambda b: (0, 0)),
        ],
        out_specs=[
            pl.BlockSpec((1, 4, P_PAD), lambda b: (b, 0, 0)),
            pl.BlockSpec((1, 1, P_PAD), lambda b: (b, 0, 0)),
        ],
        out_shape=[
            jax.ShapeDtypeStruct((B, 4, P_PAD), jnp.float32),
            jax.ShapeDtypeStruct((B, 1, P_PAD), jnp.int32),
        ],
        compiler_params=pltpu.CompilerParams(
            dimension_semantics=("parallel",),
            vmem_limit_bytes=64 * 1024 * 1024,
        ),
        name="ssd_target_transform",
    )(gt_boxes, vals, table)
    locations = jnp.transpose(loc, (0, 2, 1))[:, :P_REAL, :]
    labels = lab[:, 0, :P_REAL]
    return locations, labels


# speedup vs baseline: 58.4221x; 1.0334x over previous
import jax
import jax.numpy as jnp
from jax import lax
from jax.experimental import pallas as pl
from jax.experimental.pallas import tpu as pltpu

CENTER_VARIANCE = 0.1
SIZE_VARIANCE = 0.2
IOU_THRESHOLD = 0.5
EPS = 1e-5
P_REAL = 8732
P_PAD = 8960
G = 64


def _ssd_kernel(gt_ref, lab_in_ref, tb_ref, loc_ref, lab_ref):
    gt = gt_ref[0]
    vals = jnp.concatenate(
        [jnp.transpose(gt, (1, 0)),
         lab_in_ref[0].astype(jnp.float32),
         jnp.zeros((3, G), jnp.float32)], axis=0)
    pcx = tb_ref[0:1, :]
    pcy = tb_ref[1:2, :]
    pw = tb_ref[2:3, :]
    ph = tb_ref[3:4, :]
    px1 = tb_ref[4:5, :]
    py1 = tb_ref[5:6, :]
    px2 = tb_ref[6:7, :]
    py2 = tb_ref[7:8, :]
    pa = tb_ref[8:9, :]
    pwv = tb_ref[9:10, :]
    phv = tb_ref[10:11, :]
    p_iota = tb_ref[11:12, :]
    g_iota = lax.broadcasted_iota(jnp.int32, (G, 1), 0).astype(jnp.float32)
    gx1 = gt[:, 0:1]
    gy1 = gt[:, 1:2]
    gx2 = gt[:, 2:3]
    gy2 = gt[:, 3:4]
    ow = jnp.maximum(jnp.minimum(gx2, px2) - jnp.maximum(gx1, px1), 0.0)
    oh = jnp.maximum(jnp.minimum(gy2, py2) - jnp.maximum(gy1, py1), 0.0)
    overlap = ow * oh
    ga = jnp.maximum(gx2 - gx1, 0.0) * jnp.maximum(gy2 - gy1, 0.0)
    iou = overlap / (ga + pa - overlap + EPS)
    best = jnp.max(iou, axis=0, keepdims=True)
    bt_idx = jnp.min(jnp.where(iou == best, g_iota, float(G)),
                     axis=0, keepdims=True)
    bp_val = jnp.max(iou, axis=1, keepdims=True)
    bppt = jnp.min(jnp.where(iou == bp_val, p_iota, float(P_PAD)),
                   axis=1, keepdims=True)
    forced = jnp.max(jnp.where(bppt == p_iota, g_iota, -1.0),
                     axis=0, keepdims=True)
    new_idx = jnp.where(forced >= 0.0, forced, bt_idx)
    best = jnp.where(forced >= 0.0, 2.0, best)
    onehot = (g_iota == new_idx).astype(jnp.float32)
    out8 = lax.dot_general(vals, onehot, (((1,), (0,)), ((), ())),
                           preferred_element_type=jnp.float32)
    x1, y1, x2, y2 = out8[0:1], out8[1:2], out8[2:3], out8[3:4]
    labf = out8[4:5]
    cx = (x1 + x2) / 2
    cy = (y1 + y2) / 2
    w = x2 - x1
    h = y2 - y1
    loc_x = (cx - pcx) / pwv
    loc_y = (cy - pcy) / phv
    loc_w = jnp.log(w / pw) / SIZE_VARIANCE
    loc_h = jnp.log(h / ph) / SIZE_VARIANCE
    loc4 = jnp.concatenate([loc_x, loc_y, loc_w, loc_h], axis=0)
    labv = jnp.where(best < IOU_THRESHOLD, 0, labf.astype(jnp.int32))
    loc_ref[0] = loc4[:, :P_REAL]
    lab_ref[0] = labv[:, :P_REAL]


def kernel(gt_boxes, gt_labels, priors_center):
    B = gt_boxes.shape[0]
    pad = jnp.tile(jnp.array([[-10.0, -10.0, 1.0, 1.0]], jnp.float32),
                   (P_PAD - P_REAL, 1))
    pc = jnp.concatenate([priors_center.astype(jnp.float32), pad], axis=0)
    pcx, pcy, pw, ph = pc[:, 0], pc[:, 1], pc[:, 2], pc[:, 3]
    px1 = pcx - pw / 2
    py1 = pcy - ph / 2
    px2 = pcx + pw / 2
    py2 = pcy + ph / 2
    pa = jnp.maximum(px2 - px1, 0.0) * jnp.maximum(py2 - py1, 0.0)
    table = jnp.stack([pcx, pcy, pw, ph, px1, py1, px2, py2, pa,
                       pw * CENTER_VARIANCE, ph * CENTER_VARIANCE,
                       jnp.arange(P_PAD, dtype=jnp.float32)], axis=0)
    loc, lab = pl.pallas_call(
        _ssd_kernel,
        grid=(B,),
        in_specs=[
            pl.BlockSpec((1, G, 4), lambda b: (b, 0, 0)),
            pl.BlockSpec((1, 1, G), lambda b: (b, 0, 0)),
            pl.BlockSpec((12, P_PAD), lambda b: (0, 0)),
        ],
        out_specs=[
            pl.BlockSpec((1, 4, P_REAL), lambda b: (b, 0, 0)),
            pl.BlockSpec((1, 1, P_REAL), lambda b: (b, 0, 0)),
        ],
        out_shape=[
            jax.ShapeDtypeStruct((B, 4, P_REAL), jnp.float32),
            jax.ShapeDtypeStruct((B, 1, P_REAL), jnp.int32),
        ],
        compiler_params=pltpu.CompilerParams(
            dimension_semantics=("parallel",),
            vmem_limit_bytes=64 * 1024 * 1024,
        ),
        name="ssd_target_transform",
    )(gt_boxes, gt_labels[:, None, :], table)
    locations = jnp.transpose(loc, (0, 2, 1))
    labels = lab[:, 0, :]
    return locations, labels


# speedup vs baseline: 64.7221x; 1.1078x over previous
import jax
import jax.numpy as jnp
from jax import lax
from jax.experimental import pallas as pl
from jax.experimental.pallas import tpu as pltpu

CENTER_VARIANCE = 0.1
SIZE_VARIANCE = 0.2
IOU_THRESHOLD = 0.5
EPS = 1e-5
P_REAL = 8732
P_PAD = 8960
G = 64
NCH = 35
CH = P_PAD // NCH


def _ssd_kernel(gt_ref, lab_in_ref, tb_ref, loc_ref, lab_ref):
    gt = gt_ref[0]
    vals = jnp.concatenate(
        [jnp.transpose(gt, (1, 0)),
         lab_in_ref[0].astype(jnp.float32),
         jnp.zeros((3, G), jnp.float32)], axis=0)
    g_iota = lax.broadcasted_iota(jnp.int32, (G, 1), 0).astype(jnp.float32)
    gx1 = gt[:, 0:1]
    gy1 = gt[:, 1:2]
    gx2 = gt[:, 2:3]
    gy2 = gt[:, 3:4]
    ga = jnp.maximum(gx2 - gx1, 0.0) * jnp.maximum(gy2 - gy1, 0.0)
    run_bpv = jnp.full((G, 1), -1.0, jnp.float32)
    run_bppt = jnp.full((G, 1), float(P_PAD), jnp.float32)
    best_l, bt_l = [], []
    for c in range(NCH):
        if c * CH >= P_REAL:
            break
        s = slice(c * CH, (c + 1) * CH)
        px1 = tb_ref[4:5, s]
        py1 = tb_ref[5:6, s]
        px2 = tb_ref[6:7, s]
        py2 = tb_ref[7:8, s]
        pa = tb_ref[8:9, s]
        p_io = tb_ref[11:12, s]
        ow = jnp.maximum(jnp.minimum(gx2, px2) - jnp.maximum(gx1, px1), 0.0)
        oh = jnp.maximum(jnp.minimum(gy2, py2) - jnp.maximum(gy1, py1), 0.0)
        overlap = ow * oh
        iou = overlap / (ga + pa - overlap + EPS)
        best_c = jnp.max(iou, axis=0, keepdims=True)
        bt_c = jnp.min(jnp.where(iou == best_c, g_iota, float(G)),
                       axis=0, keepdims=True)
        bpv_c = jnp.max(iou, axis=1, keepdims=True)
        bppt_c = jnp.min(jnp.where(iou == bpv_c, p_io, float(P_PAD)),
                         axis=1, keepdims=True)
        take = bpv_c > run_bpv
        run_bppt = jnp.where(take, bppt_c, run_bppt)
        run_bpv = jnp.where(take, bpv_c, run_bpv)
        best_l.append(best_c)
        bt_l.append(bt_c)
    for c in range(NCH):
        if c * CH >= P_REAL:
            break
        s = slice(c * CH, (c + 1) * CH)
        wout = min(P_REAL - c * CH, CH)
        pcx = tb_ref[0:1, s]
        pcy = tb_ref[1:2, s]
        pw = tb_ref[2:3, s]
        ph = tb_ref[3:4, s]
        pwv = tb_ref[9:10, s]
        phv = tb_ref[10:11, s]
        p_io = tb_ref[11:12, s]
        forced = jnp.max(jnp.where(run_bppt == p_io, g_iota, -1.0),
                         axis=0, keepdims=True)
        new_idx = jnp.where(forced >= 0.0, forced, bt_l[c])
        best = jnp.where(forced >= 0.0, 2.0, best_l[c])
        onehot = (g_iota == new_idx).astype(jnp.float32)
        out8 = lax.dot_general(vals, onehot, (((1,), (0,)), ((), ())),
                               preferred_element_type=jnp.float32)
        x1, y1, x2, y2 = out8[0:1], out8[1:2], out8[2:3], out8[3:4]
        labf = out8[4:5]
        cx = (x1 + x2) / 2
        cy = (y1 + y2) / 2
        w = x2 - x1
        h = y2 - y1
        loc_x = (cx - pcx) / pwv
        loc_y = (cy - pcy) / phv
        loc_w = jnp.log(w / pw) / SIZE_VARIANCE
        loc_h = jnp.log(h / ph) / SIZE_VARIANCE
        loc4 = jnp.concatenate([loc_x, loc_y, loc_w, loc_h], axis=0)
        labv = jnp.where(best < IOU_THRESHOLD, 0, labf.astype(jnp.int32))
        loc_ref[0, :, c * CH:c * CH + wout] = loc4[:, :wout]
        lab_ref[0, :, c * CH:c * CH + wout] = labv[:, :wout]


def kernel(gt_boxes, gt_labels, priors_center):
    B = gt_boxes.shape[0]
    pad = jnp.tile(jnp.array([[-10.0, -10.0, 1.0, 1.0]], jnp.float32),
                   (P_PAD - P_REAL, 1))
    pc = jnp.concatenate([priors_center.astype(jnp.float32), pad], axis=0)
    pcx, pcy, pw, ph = pc[:, 0], pc[:, 1], pc[:, 2], pc[:, 3]
    px1 = pcx - pw / 2
    py1 = pcy - ph / 2
    px2 = pcx + pw / 2
    py2 = pcy + ph / 2
    pa = jnp.maximum(px2 - px1, 0.0) * jnp.maximum(py2 - py1, 0.0)
    table = jnp.stack([pcx, pcy, pw, ph, px1, py1, px2, py2, pa,
                       pw * CENTER_VARIANCE, ph * CENTER_VARIANCE,
                       jnp.arange(P_PAD, dtype=jnp.float32)], axis=0)
    loc, lab = pl.pallas_call(
        _ssd_kernel,
        grid=(B,),
        in_specs=[
            pl.BlockSpec((1, G, 4), lambda b: (b, 0, 0)),
            pl.BlockSpec((1, 1, G), lambda b: (b, 0, 0)),
            pl.BlockSpec((12, P_PAD), lambda b: (0, 0)),
        ],
        out_specs=[
            pl.BlockSpec((1, 4, P_REAL), lambda b: (b, 0, 0)),
            pl.BlockSpec((1, 1, P_REAL), lambda b: (b, 0, 0)),
        ],
        out_shape=[
            jax.ShapeDtypeStruct((B, 4, P_REAL), jnp.float32),
            jax.ShapeDtypeStruct((B, 1, P_REAL), jnp.int32),
        ],
        compiler_params=pltpu.CompilerParams(
            dimension_semantics=("parallel",),
            vmem_limit_bytes=64 * 1024 * 1024,
        ),
        name="ssd_target_transform",
    )(gt_boxes, gt_labels[:, None, :], table)
    locations = jnp.transpose(loc, (0, 2, 1))
    labels = lab[:, 0, :]
    return locations, labels
